# baseline (speedup 1.0000x reference)
import jax
import jax.numpy as jnp
from jax import lax
from jax.experimental import pallas as pl
from jax.experimental.pallas import tpu as pltpu
from jax.experimental.pallas import tpu_sc as plsc

V = 1000000
D = 64
B = 16384
F = 26

NC = 2
NS = 16
NW = NC * NS
SPW = B // NW
CHUNK = 4
G = CHUNK * F
NCH = SPW // CHUNK
NIDX = NCH * G

TBLK = 512
TGRID = (V + TBLK - 1) // TBLK
ROWS_PAD = TGRID * TBLK // 2


def _tr_body(tv_ref, out_ref):
    blk = tv_ref[...]
    tp = jnp.swapaxes(blk, 0, 1)
    out_ref[...] = jnp.concatenate(
        [tp[:TBLK // 2, :], tp[TBLK // 2:, :]], axis=1)


@jax.jit
def _repack(tv):
    return pl.pallas_call(
        _tr_body,
        grid=(TGRID,),
        in_specs=[pl.BlockSpec((D, TBLK), lambda i: (0, i))],
        out_specs=pl.BlockSpec((TBLK // 2, 128), lambda i: (i, 0)),
        out_shape=jax.ShapeDtypeStruct((ROWS_PAD, 128), jnp.float32),
    )(tv)


def _sc_body(idx_hbm, idx2_hbm, table_hbm, lin_hbm, out_inter, out_lin,
             idx_v, idx2_v, buf_a, buf_b, out_v, lin_acc,
             sem_a, sem_b, sem_l):
    wid = lax.axis_index("s") * NC + lax.axis_index("c")

    pltpu.sync_copy(idx_hbm.at[wid], idx_v)
    pltpu.sync_copy(idx2_hbm.at[wid], idx2_v)

    def row_copy(j, buf, sem):
        return pltpu.make_async_copy(
            table_hbm.at[idx2_v.at[pl.ds(j * G, G)]], buf, sem)

    def lin_copy(j):
        return pltpu.make_async_copy(
            lin_hbm.at[idx_v.at[pl.ds(j * G, G)]], lin_acc.at[j], sem_l)

    row_copy(0, buf_a, sem_a).start()
    lin_copy(0).start()
    row_copy(1, buf_b, sem_b).start()
    lin_copy(1).start()

    def step(t, carry):
        for slot, (buf, sem) in enumerate(((buf_a, sem_a), (buf_b, sem_b))):
            j = 2 * t + slot
            row_copy(j, buf, sem).wait()
            lin_copy(j).wait()
            for s in range(CHUNK):
                acc = [jnp.zeros((16,), jnp.float32) for _ in range(4)]
                accq = [jnp.zeros((16,), jnp.float32) for _ in range(4)]
                for r in range(F):
                    row = s * F + r
                    for c in range(4):
                        v = buf[row, pl.ds(16 * c, 16)]
                        acc[c] = acc[c] + v
                        accq[c] = accq[c] + v * v
                orow = CHUNK * j + s
                for c in range(4):
                    out_v[orow, pl.ds(16 * c, 16)] = 0.5 * (
                        acc[c] * acc[c] - accq[c])
            nxt = j + 2

            @pl.when(nxt < NCH)
            def _():
                row_copy(nxt, buf, sem).start()
                lin_copy(nxt).start()
        return carry

    lax.fori_loop(0, NCH // 2, step, 0)

    pltpu.sync_copy(out_v, out_inter.at[pl.ds(wid * SPW, SPW)])
    pltpu.sync_copy(lin_acc, out_lin.at[wid])


@jax.jit
def _sc_gather(idx, idx2, table, lin_tab):
    mesh = plsc.VectorSubcoreMesh(core_axis_name="c", subcore_axis_name="s")
    f = pl.kernel(
        _sc_body,
        mesh=mesh,
        compiler_params=pltpu.CompilerParams(use_tc_tiling_on_sc=False),
        out_type=[
            jax.ShapeDtypeStruct((B, D), jnp.float32),
            jax.ShapeDtypeStruct((NW, NCH, G), jnp.float32),
        ],
        scratch_types=[
            pltpu.VMEM((NIDX,), jnp.int32),
            pltpu.VMEM((NIDX,), jnp.int32),
            pltpu.VMEM((G, D), jnp.float32),
            pltpu.VMEM((G, D), jnp.float32),
            pltpu.VMEM((SPW, D), jnp.float32),
            pltpu.VMEM((NCH, G), jnp.float32),
            pltpu.SemaphoreType.DMA,
            pltpu.SemaphoreType.DMA,
            pltpu.SemaphoreType.DMA,
        ],
    )
    return f(idx, idx2, table, lin_tab)


def _mlp_body(inter_ref, lin_ref, w1_ref, b1_ref, w2_ref, b2_ref, w3_ref,
              c_ref, out_ref):
    inter = inter_ref[...]
    h = jnp.dot(inter, w1_ref[...], preferred_element_type=jnp.float32)
    h = jnp.maximum(h + b1_ref[...], 0.0)
    h = jnp.dot(h, w2_ref[...], preferred_element_type=jnp.float32)
    h = jnp.maximum(h + b2_ref[...], 0.0)
    deep = jnp.sum(h * w3_ref[...], axis=1, keepdims=True)
    lr = jnp.sum(lin_ref[...], axis=1, keepdims=True)
    out_ref[...] = jax.nn.sigmoid(deep + lr + c_ref[...])


@jax.jit
def _mlp(inter, lin2, W1, b1r, W2, b2r, w3r, c):
    blk = 2048
    return pl.pallas_call(
        _mlp_body,
        grid=(B // blk,),
        in_specs=[
            pl.BlockSpec((blk, D), lambda i: (i, 0)),
            pl.BlockSpec((blk, F), lambda i: (i, 0)),
            pl.BlockSpec((D, 256), lambda i: (0, 0)),
            pl.BlockSpec((1, 256), lambda i: (0, 0)),
            pl.BlockSpec((256, 128), lambda i: (0, 0)),
            pl.BlockSpec((1, 128), lambda i: (0, 0)),
            pl.BlockSpec((1, 128), lambda i: (0, 0)),
            pl.BlockSpec((1, 1), lambda i: (0, 0)),
        ],
        out_specs=pl.BlockSpec((blk, 1), lambda i: (i, 0)),
        out_shape=jax.ShapeDtypeStruct((B, 1), jnp.float32),
    )(inter, lin2, W1, b1r, W2, b2r, w3r, c)


def kernel(x, emb_linear, emb_table, bias, W1, b1, W2, b2, W3, b3):
    g = x.astype(jnp.int32)
    idx = g.reshape(NW, NIDX)
    v = g & (TBLK - 1)
    s = g - v + ((v & (TBLK // 2 - 1)) << 1) + (v >> 8)
    idx2 = s.reshape(NW, NIDX)
    lin_tab = emb_linear.reshape(V)
    t1 = _repack(emb_table.T)
    tbl = t1.reshape(2 * ROWS_PAD, D)
    inter, lin_vals = _sc_gather(idx, idx2, tbl, lin_tab)
    lin2 = lin_vals.reshape(B, F)
    c = (b3 + bias).reshape(1, 1)
    return _mlp(inter, lin2, W1, b1.reshape(1, 256), W2,
                b2.reshape(1, 128), W3.reshape(1, 128), c)


# speedup vs baseline: 1.9158x; 1.9158x over previous
import jax
import jax.numpy as jnp
from jax import lax
from jax.experimental import pallas as pl
from jax.experimental.pallas import tpu as pltpu
from jax.experimental.pallas import tpu_sc as plsc

V = 1000000
D = 64
B = 16384
F = 26

NC = 2
NS = 16
NW = NC * NS
SPW = B // NW
CHUNK = 4
G = CHUNK * F
NCH = SPW // CHUNK
NIDX = NCH * G


def _sc_body(idx_hbm, table_hbm, lin_hbm, out_inter, out_lin,
             idx_v, buf_a, buf_b, out_v, lin_acc, sem_a, sem_b, sem_l):
    wid = lax.axis_index("s") * NC + lax.axis_index("c")

    pltpu.sync_copy(idx_hbm.at[wid], idx_v)

    def row_copy(j, buf, sem):
        return pltpu.make_async_copy(
            table_hbm.at[idx_v.at[pl.ds(j * G, G)]], buf, sem)

    def lin_copy(j):
        return pltpu.make_async_copy(
            lin_hbm.at[idx_v.at[pl.ds(j * G, G)]], lin_acc.at[j], sem_l)

    row_copy(0, buf_a, sem_a).start()
    lin_copy(0).start()
    row_copy(1, buf_b, sem_b).start()
    lin_copy(1).start()

    def step(t, carry):
        for slot, (buf, sem) in enumerate(((buf_a, sem_a), (buf_b, sem_b))):
            j = 2 * t + slot
            row_copy(j, buf, sem).wait()
            lin_copy(j).wait()
            for s in range(CHUNK):
                acc = [jnp.zeros((16,), jnp.float32) for _ in range(4)]
                accq = [jnp.zeros((16,), jnp.float32) for _ in range(4)]
                for r in range(F):
                    row = s * F + r
                    for c in range(4):
                        v = buf[row, pl.ds(16 * c, 16)]
                        acc[c] = acc[c] + v
                        accq[c] = accq[c] + v * v
                orow = CHUNK * j + s
                for c in range(4):
                    out_v[orow, pl.ds(16 * c, 16)] = 0.5 * (
                        acc[c] * acc[c] - accq[c])
            nxt = j + 2

            @pl.when(nxt < NCH)
            def _():
                row_copy(nxt, buf, sem).start()
                lin_copy(nxt).start()
        return carry

    lax.fori_loop(0, NCH // 2, step, 0)

    pltpu.sync_copy(out_v, out_inter.at[pl.ds(wid * SPW, SPW)])
    pltpu.sync_copy(lin_acc, out_lin.at[wid])


@jax.jit
def _sc_gather(idx, table, lin_tab):
    mesh = plsc.VectorSubcoreMesh(core_axis_name="c", subcore_axis_name="s")
    f = pl.kernel(
        _sc_body,
        mesh=mesh,
        compiler_params=pltpu.CompilerParams(use_tc_tiling_on_sc=False),
        out_type=[
            jax.ShapeDtypeStruct((B, D), jnp.float32),
            jax.ShapeDtypeStruct((NW, NCH, G), jnp.float32),
        ],
        scratch_types=[
            pltpu.VMEM((NIDX,), jnp.int32),
            pltpu.VMEM((G, D), jnp.float32),
            pltpu.VMEM((G, D), jnp.float32),
            pltpu.VMEM((SPW, D), jnp.float32),
            pltpu.VMEM((NCH, G), jnp.float32),
            pltpu.SemaphoreType.DMA,
            pltpu.SemaphoreType.DMA,
            pltpu.SemaphoreType.DMA,
        ],
    )
    return f(idx, table, lin_tab)


def _mlp_body(inter_ref, lin_ref, w1_ref, b1_ref, w2_ref, b2_ref, w3_ref,
              c_ref, out_ref):
    inter = inter_ref[...]
    h = jnp.dot(inter, w1_ref[...], preferred_element_type=jnp.float32)
    h = jnp.maximum(h + b1_ref[...], 0.0)
    h = jnp.dot(h, w2_ref[...], preferred_element_type=jnp.float32)
    h = jnp.maximum(h + b2_ref[...], 0.0)
    deep = jnp.sum(h * w3_ref[...], axis=1, keepdims=True)
    lr = jnp.sum(lin_ref[...], axis=1, keepdims=True)
    out_ref[...] = jax.nn.sigmoid(deep + lr + c_ref[...])


@jax.jit
def _mlp(inter, lin2, W1, b1r, W2, b2r, w3r, c):
    blk = 2048
    return pl.pallas_call(
        _mlp_body,
        grid=(B // blk,),
        in_specs=[
            pl.BlockSpec((blk, D), lambda i: (i, 0)),
            pl.BlockSpec((blk, F), lambda i: (i, 0)),
            pl.BlockSpec((D, 256), lambda i: (0, 0)),
            pl.BlockSpec((1, 256), lambda i: (0, 0)),
            pl.BlockSpec((256, 128), lambda i: (0, 0)),
            pl.BlockSpec((1, 128), lambda i: (0, 0)),
            pl.BlockSpec((1, 128), lambda i: (0, 0)),
            pl.BlockSpec((1, 1), lambda i: (0, 0)),
        ],
        out_specs=pl.BlockSpec((blk, 1), lambda i: (i, 0)),
        out_shape=jax.ShapeDtypeStruct((B, 1), jnp.float32),
    )(inter, lin2, W1, b1r, W2, b2r, w3r, c)


def kernel(x, emb_linear, emb_table, bias, W1, b1, W2, b2, W3, b3):
    idx = x.astype(jnp.int32).reshape(NW, NIDX)
    lin_tab = emb_linear.reshape(V)
    inter, lin_vals = _sc_gather(idx, emb_table, lin_tab)
    lin2 = lin_vals.reshape(B, F)
    c = (b3 + bias).reshape(1, 1)
    return _mlp(inter, lin2, W1, b1.reshape(1, 256), W2,
                b2.reshape(1, 128), W3.reshape(1, 128), c)


# speedup vs baseline: 2.1446x; 1.1194x over previous
import jax
import jax.numpy as jnp
from jax import lax
from jax.experimental import pallas as pl
from jax.experimental.pallas import tpu as pltpu
from jax.experimental.pallas import tpu_sc as plsc

V = 1000000
D = 64
B = 16384
F = 26

NC = 2
NS = 16
NW = NC * NS
SPW = B // NW
CHUNK = 4
G = CHUNK * F
NCH = SPW // CHUNK
NIDX = NCH * G

TBLK = 2048
TGRID = (V + TBLK - 1) // TBLK
ROWS_PAD = TGRID * TBLK // 2


def _tr_body(tv_ref, out_ref):
    blk = tv_ref[...]
    tp = jnp.swapaxes(blk, 0, 1)
    out_ref[...] = jnp.concatenate(
        [tp[:TBLK // 2, :], tp[TBLK // 2:, :]], axis=1)


@jax.jit
def _repack(tv):
    return pl.pallas_call(
        _tr_body,
        grid=(TGRID,),
        in_specs=[pl.BlockSpec((D, TBLK), lambda i: (0, i))],
        out_specs=pl.BlockSpec((TBLK // 2, 128), lambda i: (i, 0)),
        out_shape=jax.ShapeDtypeStruct((ROWS_PAD, 128), jnp.float32),
    )(tv)


def _sc_body(idx_hbm, idx2_hbm, table_hbm, lin_hbm, out_inter, out_lin,
             idx_v, idx2_v, buf_a, buf_b, out_v, lin_acc,
             sem_a, sem_b, sem_l):
    wid = lax.axis_index("s") * NC + lax.axis_index("c")

    pltpu.sync_copy(idx_hbm.at[wid], idx_v)
    pltpu.sync_copy(idx2_hbm.at[wid], idx2_v)

    def row_copy(j, buf, sem):
        return pltpu.make_async_copy(
            table_hbm.at[idx2_v.at[pl.ds(j * G, G)]], buf, sem)

    def lin_copy(j):
        return pltpu.make_async_copy(
            lin_hbm.at[idx_v.at[pl.ds(j * G, G)]], lin_acc.at[j], sem_l)

    row_copy(0, buf_a, sem_a).start()
    lin_copy(0).start()
    row_copy(1, buf_b, sem_b).start()
    lin_copy(1).start()

    def step(t, carry):
        for slot, (buf, sem) in enumerate(((buf_a, sem_a), (buf_b, sem_b))):
            j = 2 * t + slot
            row_copy(j, buf, sem).wait()
            lin_copy(j).wait()
            for s in range(CHUNK):
                acc = [jnp.zeros((16,), jnp.float32) for _ in range(4)]
                accq = [jnp.zeros((16,), jnp.float32) for _ in range(4)]
                for r in range(F):
                    row = s * F + r
                    for c in range(4):
                        v = buf[row, pl.ds(16 * c, 16)]
                        acc[c] = acc[c] + v
                        accq[c] = accq[c] + v * v
                orow = CHUNK * j + s
                for c in range(4):
                    out_v[orow, pl.ds(16 * c, 16)] = 0.5 * (
                        acc[c] * acc[c] - accq[c])
            nxt = j + 2

            @pl.when(nxt < NCH)
            def _():
                row_copy(nxt, buf, sem).start()
                lin_copy(nxt).start()
        return carry

    lax.fori_loop(0, NCH // 2, step, 0)

    pltpu.sync_copy(out_v, out_inter.at[pl.ds(wid * SPW, SPW)])
    pltpu.sync_copy(lin_acc, out_lin.at[wid])


@jax.jit
def _sc_gather(idx, idx2, table, lin_tab):
    mesh = plsc.VectorSubcoreMesh(core_axis_name="c", subcore_axis_name="s")
    f = pl.kernel(
        _sc_body,
        mesh=mesh,
        compiler_params=pltpu.CompilerParams(use_tc_tiling_on_sc=False),
        out_type=[
            jax.ShapeDtypeStruct((B, D), jnp.float32),
            jax.ShapeDtypeStruct((NW, NCH, G), jnp.float32),
        ],
        scratch_types=[
            pltpu.VMEM((NIDX,), jnp.int32),
            pltpu.VMEM((NIDX,), jnp.int32),
            pltpu.VMEM((G, D), jnp.float32),
            pltpu.VMEM((G, D), jnp.float32),
            pltpu.VMEM((SPW, D), jnp.float32),
            pltpu.VMEM((NCH, G), jnp.float32),
            pltpu.SemaphoreType.DMA,
            pltpu.SemaphoreType.DMA,
            pltpu.SemaphoreType.DMA,
        ],
    )
    return f(idx, idx2, table, lin_tab)


def _mlp_body(inter_ref, lin_ref, w1_ref, b1_ref, w2_ref, b2_ref, w3_ref,
              c_ref, out_ref):
    inter = inter_ref[...]
    h = jnp.dot(inter, w1_ref[...], preferred_element_type=jnp.float32)
    h = jnp.maximum(h + b1_ref[...], 0.0)
    h = jnp.dot(h, w2_ref[...], preferred_element_type=jnp.float32)
    h = jnp.maximum(h + b2_ref[...], 0.0)
    deep = jnp.sum(h * w3_ref[...], axis=1, keepdims=True)
    lr = jnp.sum(lin_ref[...], axis=1, keepdims=True)
    out_ref[...] = jax.nn.sigmoid(deep + lr + c_ref[...])


@jax.jit
def _mlp(inter, lin2, W1, b1r, W2, b2r, w3r, c):
    blk = 2048
    return pl.pallas_call(
        _mlp_body,
        grid=(B // blk,),
        in_specs=[
            pl.BlockSpec((blk, D), lambda i: (i, 0)),
            pl.BlockSpec((blk, F), lambda i: (i, 0)),
            pl.BlockSpec((D, 256), lambda i: (0, 0)),
            pl.BlockSpec((1, 256), lambda i: (0, 0)),
            pl.BlockSpec((256, 128), lambda i: (0, 0)),
            pl.BlockSpec((1, 128), lambda i: (0, 0)),
            pl.BlockSpec((1, 128), lambda i: (0, 0)),
            pl.BlockSpec((1, 1), lambda i: (0, 0)),
        ],
        out_specs=pl.BlockSpec((blk, 1), lambda i: (i, 0)),
        out_shape=jax.ShapeDtypeStruct((B, 1), jnp.float32),
    )(inter, lin2, W1, b1r, W2, b2r, w3r, c)


def kernel(x, emb_linear, emb_table, bias, W1, b1, W2, b2, W3, b3):
    g = x.astype(jnp.int32)
    idx = g.reshape(NW, NIDX)
    v = g & (TBLK - 1)
    s = g - v + ((v & (TBLK // 2 - 1)) << 1) + (v >> 10)
    idx2 = s.reshape(NW, NIDX)
    lin_tab = emb_linear.reshape(V)
    t1 = _repack(emb_table.T)
    tbl = t1.reshape(2 * ROWS_PAD, D)
    inter, lin_vals = _sc_gather(idx, idx2, tbl, lin_tab)
    lin2 = lin_vals.reshape(B, F)
    c = (b3 + bias).reshape(1, 1)
    return _mlp(inter, lin2, W1, b1.reshape(1, 256), W2,
                b2.reshape(1, 128), W3.reshape(1, 128), c)


# speedup vs baseline: 2.6212x; 1.2223x over previous
import jax
import jax.numpy as jnp
from jax import lax
from jax.experimental import pallas as pl
from jax.experimental.pallas import tpu as pltpu
from jax.experimental.pallas import tpu_sc as plsc

V = 1000000
D = 64
B = 16384
F = 26

NC = 2
NS = 16
NW = NC * NS
SPW = B // NW
CHUNK = 4
G = CHUNK * F
NCH = SPW // CHUNK
NIDX = NCH * G

TBLK = 4096
TGRID = (V + TBLK - 1) // TBLK
ROWS_PAD = TGRID * TBLK // 2


def _tr_body(tv_ref, out_ref):
    blk = tv_ref[...]
    tp = jnp.swapaxes(blk, 0, 1)
    out_ref[...] = jnp.concatenate(
        [tp[:TBLK // 2, :], tp[TBLK // 2:, :]], axis=1)


@jax.jit
def _repack(tv):
    return pl.pallas_call(
        _tr_body,
        grid=(TGRID,),
        in_specs=[pl.BlockSpec((D, TBLK), lambda i: (0, i))],
        out_specs=pl.BlockSpec((TBLK // 2, 128), lambda i: (i, 0)),
        out_shape=jax.ShapeDtypeStruct((ROWS_PAD, 128), jnp.float32),
    )(tv)


def _sc_body(idx_hbm, idx2_hbm, table_hbm, lin_hbm, out_inter, out_lin,
             idx_v, idx2_v, buf_a, buf_b, out_v, lin_acc,
             sem_a, sem_b, sem_l):
    wid = lax.axis_index("s") * NC + lax.axis_index("c")

    pltpu.sync_copy(idx_hbm.at[wid], idx_v)
    pltpu.sync_copy(idx2_hbm.at[wid], idx2_v)

    def row_copy(j, buf, sem):
        return pltpu.make_async_copy(
            table_hbm.at[idx2_v.at[pl.ds(j * G, G)]], buf, sem)

    def lin_copy(j):
        return pltpu.make_async_copy(
            lin_hbm.at[idx_v.at[pl.ds(j * G, G)]], lin_acc.at[j], sem_l)

    row_copy(0, buf_a, sem_a).start()
    lin_copy(0).start()
    row_copy(1, buf_b, sem_b).start()
    lin_copy(1).start()

    def step(t, carry):
        for slot, (buf, sem) in enumerate(((buf_a, sem_a), (buf_b, sem_b))):
            j = 2 * t + slot
            row_copy(j, buf, sem).wait()
            lin_copy(j).wait()
            for s in range(CHUNK):
                acc = [jnp.zeros((16,), jnp.float32) for _ in range(4)]
                accq = [jnp.zeros((16,), jnp.float32) for _ in range(4)]
                for r in range(F):
                    row = s * F + r
                    for c in range(4):
                        v = buf[row, pl.ds(16 * c, 16)]
                        acc[c] = acc[c] + v
                        accq[c] = accq[c] + v * v
                orow = CHUNK * j + s
                for c in range(4):
                    out_v[orow, pl.ds(16 * c, 16)] = 0.5 * (
                        acc[c] * acc[c] - accq[c])
            nxt = j + 2

            @pl.when(nxt < NCH)
            def _():
                row_copy(nxt, buf, sem).start()
                lin_copy(nxt).start()
        return carry

    lax.fori_loop(0, NCH // 2, step, 0)

    pltpu.sync_copy(out_v, out_inter.at[pl.ds(wid * SPW, SPW)])
    pltpu.sync_copy(lin_acc, out_lin.at[wid])


@jax.jit
def _sc_gather(idx, idx2, table, lin_tab):
    mesh = plsc.VectorSubcoreMesh(core_axis_name="c", subcore_axis_name="s")
    f = pl.kernel(
        _sc_body,
        mesh=mesh,
        compiler_params=pltpu.CompilerParams(use_tc_tiling_on_sc=False),
        out_type=[
            jax.ShapeDtypeStruct((B, D), jnp.float32),
            jax.ShapeDtypeStruct((NW, NCH, G), jnp.float32),
        ],
        scratch_types=[
            pltpu.VMEM((NIDX,), jnp.int32),
            pltpu.VMEM((NIDX,), jnp.int32),
            pltpu.VMEM((G, D), jnp.float32),
            pltpu.VMEM((G, D), jnp.float32),
            pltpu.VMEM((SPW, D), jnp.float32),
            pltpu.VMEM((NCH, G), jnp.float32),
            pltpu.SemaphoreType.DMA,
            pltpu.SemaphoreType.DMA,
            pltpu.SemaphoreType.DMA,
        ],
    )
    return f(idx, idx2, table, lin_tab)


def _mlp_body(inter_ref, lin_ref, w1_ref, b1_ref, w2_ref, b2_ref, w3_ref,
              c_ref, out_ref):
    inter = inter_ref[...]
    h = jnp.dot(inter, w1_ref[...], preferred_element_type=jnp.float32)
    h = jnp.maximum(h + b1_ref[...], 0.0)
    h = jnp.dot(h, w2_ref[...], preferred_element_type=jnp.float32)
    h = jnp.maximum(h + b2_ref[...], 0.0)
    deep = jnp.sum(h * w3_ref[...], axis=1, keepdims=True)
    lr = jnp.sum(lin_ref[...], axis=1, keepdims=True)
    out_ref[...] = jax.nn.sigmoid(deep + lr + c_ref[...])


@jax.jit
def _mlp(inter, lin2, W1, b1r, W2, b2r, w3r, c):
    blk = 2048
    return pl.pallas_call(
        _mlp_body,
        grid=(B // blk,),
        in_specs=[
            pl.BlockSpec((blk, D), lambda i: (i, 0)),
            pl.BlockSpec((blk, F), lambda i: (i, 0)),
            pl.BlockSpec((D, 256), lambda i: (0, 0)),
            pl.BlockSpec((1, 256), lambda i: (0, 0)),
            pl.BlockSpec((256, 128), lambda i: (0, 0)),
            pl.BlockSpec((1, 128), lambda i: (0, 0)),
            pl.BlockSpec((1, 128), lambda i: (0, 0)),
            pl.BlockSpec((1, 1), lambda i: (0, 0)),
        ],
        out_specs=pl.BlockSpec((blk, 1), lambda i: (i, 0)),
        out_shape=jax.ShapeDtypeStruct((B, 1), jnp.float32),
    )(inter, lin2, W1, b1r, W2, b2r, w3r, c)


def kernel(x, emb_linear, emb_table, bias, W1, b1, W2, b2, W3, b3):
    g = x.astype(jnp.int32)
    idx = g.reshape(NW, NIDX)
    v = g & (TBLK - 1)
    s = g - v + ((v & (TBLK // 2 - 1)) << 1) + (v >> (TBLK.bit_length() - 2))
    idx2 = s.reshape(NW, NIDX)
    lin_tab = emb_linear.T.reshape(V)
    t1 = _repack(emb_table.T)
    tbl = t1.reshape(2 * ROWS_PAD, D)
    inter, lin_vals = _sc_gather(idx, idx2, tbl, lin_tab)
    lin2 = lin_vals.reshape(B, F)
    c = (b3 + bias).reshape(1, 1)
    return _mlp(inter, lin2, W1, b1.reshape(1, 256), W2,
                b2.reshape(1, 128), W3.reshape(1, 128), c)


# speedup vs baseline: 2.9959x; 1.1429x over previous
import jax
import jax.numpy as jnp
from jax import lax
from jax.experimental import pallas as pl
from jax.experimental.pallas import tpu as pltpu
from jax.experimental.pallas import tpu_sc as plsc

V = 1000000
D = 64
B = 16384
F = 26

NC = 2
NS = 16
NW = NC * NS
SPW = B // NW
CHUNK = 4
G = CHUNK * F
NCH = SPW // CHUNK
NIDX = NCH * G

TBLK = 8192
TGRID = (V + TBLK - 1) // TBLK
ROWS_PAD = TGRID * TBLK // 2


def _tr_body(tv_ref, out_ref):
    blk = tv_ref[...]
    tp = jnp.swapaxes(blk, 0, 1)
    out_ref[...] = jnp.concatenate(
        [tp[:TBLK // 2, :], tp[TBLK // 2:, :]], axis=1)


@jax.jit
def _repack(tv):
    return pl.pallas_call(
        _tr_body,
        grid=(TGRID,),
        in_specs=[pl.BlockSpec((D, TBLK), lambda i: (0, i))],
        out_specs=pl.BlockSpec((TBLK // 2, 128), lambda i: (i, 0)),
        out_shape=jax.ShapeDtypeStruct((ROWS_PAD, 128), jnp.float32),
    )(tv)


def _sc_body(idx_hbm, idx2_hbm, table_hbm, lin_hbm, out_inter, out_lin,
             idx_v, idx2_v, buf_a, buf_b, out_v, lin_acc,
             sem_a, sem_b, sem_l):
    wid = lax.axis_index("s") * NC + lax.axis_index("c")

    pltpu.sync_copy(idx_hbm.at[wid], idx_v)
    pltpu.sync_copy(idx2_hbm.at[wid], idx2_v)

    def row_copy(j, buf, sem):
        return pltpu.make_async_copy(
            table_hbm.at[idx2_v.at[pl.ds(j * G, G)]], buf, sem)

    def lin_copy(j):
        return pltpu.make_async_copy(
            lin_hbm.at[idx_v.at[pl.ds(j * G, G)]], lin_acc.at[j], sem_l)

    row_copy(0, buf_a, sem_a).start()
    lin_copy(0).start()
    row_copy(1, buf_b, sem_b).start()
    lin_copy(1).start()

    def step(t, carry):
        for slot, (buf, sem) in enumerate(((buf_a, sem_a), (buf_b, sem_b))):
            j = 2 * t + slot
            row_copy(j, buf, sem).wait()
            lin_copy(j).wait()
            for s in range(CHUNK):
                acc = [jnp.zeros((16,), jnp.float32) for _ in range(4)]
                accq = [jnp.zeros((16,), jnp.float32) for _ in range(4)]
                for r in range(F):
                    row = s * F + r
                    for c in range(4):
                        v = buf[row, pl.ds(16 * c, 16)]
                        acc[c] = acc[c] + v
                        accq[c] = accq[c] + v * v
                orow = CHUNK * j + s
                for c in range(4):
                    out_v[orow, pl.ds(16 * c, 16)] = 0.5 * (
                        acc[c] * acc[c] - accq[c])
            nxt = j + 2

            @pl.when(nxt < NCH)
            def _():
                row_copy(nxt, buf, sem).start()
                lin_copy(nxt).start()
        return carry

    lax.fori_loop(0, NCH // 2, step, 0)

    pltpu.sync_copy(out_v, out_inter.at[pl.ds(wid * SPW, SPW)])
    pltpu.sync_copy(lin_acc, out_lin.at[wid])


@jax.jit
def _sc_gather(idx, idx2, table, lin_tab):
    mesh = plsc.VectorSubcoreMesh(core_axis_name="c", subcore_axis_name="s")
    f = pl.kernel(
        _sc_body,
        mesh=mesh,
        compiler_params=pltpu.CompilerParams(use_tc_tiling_on_sc=False),
        out_type=[
            jax.ShapeDtypeStruct((B, D), jnp.float32),
            jax.ShapeDtypeStruct((NW, NCH, G), jnp.float32),
        ],
        scratch_types=[
            pltpu.VMEM((NIDX,), jnp.int32),
            pltpu.VMEM((NIDX,), jnp.int32),
            pltpu.VMEM((G, D), jnp.float32),
            pltpu.VMEM((G, D), jnp.float32),
            pltpu.VMEM((SPW, D), jnp.float32),
            pltpu.VMEM((NCH, G), jnp.float32),
            pltpu.SemaphoreType.DMA,
            pltpu.SemaphoreType.DMA,
            pltpu.SemaphoreType.DMA,
        ],
    )
    return f(idx, idx2, table, lin_tab)


def _mlp_body(inter_ref, lin_ref, w1_ref, b1_ref, w2_ref, b2_ref, w3_ref,
              c_ref, out_ref):
    inter = inter_ref[...]
    h = jnp.dot(inter, w1_ref[...], preferred_element_type=jnp.float32)
    h = jnp.maximum(h + b1_ref[...], 0.0)
    h = jnp.dot(h, w2_ref[...], preferred_element_type=jnp.float32)
    h = jnp.maximum(h + b2_ref[...], 0.0)
    deep = jnp.sum(h * w3_ref[...], axis=1, keepdims=True)
    lr = jnp.sum(lin_ref[...], axis=1, keepdims=True)
    out_ref[...] = jax.nn.sigmoid(deep + lr + c_ref[...])


@jax.jit
def _mlp(inter, lin2, W1, b1r, W2, b2r, w3r, c):
    blk = 2048
    return pl.pallas_call(
        _mlp_body,
        grid=(B // blk,),
        in_specs=[
            pl.BlockSpec((blk, D), lambda i: (i, 0)),
            pl.BlockSpec((blk, F), lambda i: (i, 0)),
            pl.BlockSpec((D, 256), lambda i: (0, 0)),
            pl.BlockSpec((1, 256), lambda i: (0, 0)),
            pl.BlockSpec((256, 128), lambda i: (0, 0)),
            pl.BlockSpec((1, 128), lambda i: (0, 0)),
            pl.BlockSpec((1, 128), lambda i: (0, 0)),
            pl.BlockSpec((1, 1), lambda i: (0, 0)),
        ],
        out_specs=pl.BlockSpec((blk, 1), lambda i: (i, 0)),
        out_shape=jax.ShapeDtypeStruct((B, 1), jnp.float32),
    )(inter, lin2, W1, b1r, W2, b2r, w3r, c)


def kernel(x, emb_linear, emb_table, bias, W1, b1, W2, b2, W3, b3):
    g = x.astype(jnp.int32)
    idx = g.reshape(NW, NIDX)
    v = g & (TBLK - 1)
    s = g - v + ((v & (TBLK // 2 - 1)) << 1) + (v >> (TBLK.bit_length() - 2))
    idx2 = s.reshape(NW, NIDX)
    lin_tab = emb_linear.reshape(V)
    t1 = _repack(emb_table.T)
    tbl = t1.reshape(2 * ROWS_PAD, D)
    inter, lin_vals = _sc_gather(idx, idx2, tbl, lin_tab)
    lin2 = lin_vals.reshape(B, F)
    c = (b3 + bias).reshape(1, 1)
    return _mlp(inter, lin2, W1, b1.reshape(1, 256), W2,
                b2.reshape(1, 128), W3.reshape(1, 128), c)


# speedup vs baseline: 3.5438x; 1.1829x over previous
import jax
import jax.numpy as jnp
from jax import lax
from jax.experimental import pallas as pl
from jax.experimental.pallas import tpu as pltpu
from jax.experimental.pallas import tpu_sc as plsc

V = 1000000
D = 64
B = 16384
F = 26

NC = 2
NS = 16
NW = NC * NS
SPW = B // NW
CHUNK = 4
G = CHUNK * F
NCH = SPW // CHUNK
NIDX = NCH * G

TBLK = 16384
TGRID = (V + TBLK - 1) // TBLK
VPAD = TGRID * TBLK
ROWS_PAD = VPAD // 2


def _tr_body(tv_ref, lv_ref, out_ref, out2_ref):
    blk = tv_ref[...]
    tp = jnp.swapaxes(blk, 0, 1)
    out_ref[...] = jnp.concatenate(
        [tp[:TBLK // 2, :], tp[TBLK // 2:, :]], axis=1)
    lv = lv_ref[...]
    out2_ref[...] = jnp.concatenate(
        [lv[:, 128 * i:128 * (i + 1)] for i in range(TBLK // 128)], axis=0)


@jax.jit
def _repack(tv, lv):
    return pl.pallas_call(
        _tr_body,
        grid=(TGRID,),
        in_specs=[pl.BlockSpec((D, TBLK), lambda i: (0, i)),
                  pl.BlockSpec((1, TBLK), lambda i: (0, i))],
        out_specs=[pl.BlockSpec((TBLK // 2, 128), lambda i: (i, 0)),
                   pl.BlockSpec((TBLK // 128, 128), lambda i: (i, 0))],
        out_shape=[jax.ShapeDtypeStruct((ROWS_PAD, 128), jnp.float32),
                   jax.ShapeDtypeStruct((VPAD // 128, 128), jnp.float32)],
    )(tv, lv)


def _sc_body(idx_hbm, idx2_hbm, table_hbm, lin_hbm, out_inter, out_lin,
             idx_v, idx2_v, buf_a, buf_b, out_v, lin_acc,
             sem_a, sem_b, sem_l):
    wid = lax.axis_index("s") * NC + lax.axis_index("c")

    pltpu.sync_copy(idx_hbm.at[wid], idx_v)
    pltpu.sync_copy(idx2_hbm.at[wid], idx2_v)

    def row_copy(j, buf, sem):
        return pltpu.make_async_copy(
            table_hbm.at[idx2_v.at[pl.ds(j * G, G)]], buf, sem)

    def lin_copy(j):
        return pltpu.make_async_copy(
            lin_hbm.at[idx_v.at[pl.ds(j * G, G)]], lin_acc.at[j], sem_l)

    row_copy(0, buf_a, sem_a).start()
    lin_copy(0).start()
    row_copy(1, buf_b, sem_b).start()
    lin_copy(1).start()

    def step(t, carry):
        for slot, (buf, sem) in enumerate(((buf_a, sem_a), (buf_b, sem_b))):
            j = 2 * t + slot
            row_copy(j, buf, sem).wait()
            lin_copy(j).wait()
            for s in range(CHUNK):
                acc = [jnp.zeros((16,), jnp.float32) for _ in range(4)]
                accq = [jnp.zeros((16,), jnp.float32) for _ in range(4)]
                for r in range(F):
                    row = s * F + r
                    for c in range(4):
                        v = buf[row, pl.ds(16 * c, 16)]
                        acc[c] = acc[c] + v
                        accq[c] = accq[c] + v * v
                orow = CHUNK * j + s
                for c in range(4):
                    out_v[orow, pl.ds(16 * c, 16)] = 0.5 * (
                        acc[c] * acc[c] - accq[c])
            nxt = j + 2

            @pl.when(nxt < NCH)
            def _():
                row_copy(nxt, buf, sem).start()
                lin_copy(nxt).start()
        return carry

    lax.fori_loop(0, NCH // 2, step, 0)

    pltpu.sync_copy(out_v, out_inter.at[pl.ds(wid * SPW, SPW)])
    pltpu.sync_copy(lin_acc, out_lin.at[wid])


@jax.jit
def _sc_gather(idx, idx2, table, lin_tab):
    mesh = plsc.VectorSubcoreMesh(core_axis_name="c", subcore_axis_name="s")
    f = pl.kernel(
        _sc_body,
        mesh=mesh,
        compiler_params=pltpu.CompilerParams(use_tc_tiling_on_sc=False),
        out_type=[
            jax.ShapeDtypeStruct((B, D), jnp.float32),
            jax.ShapeDtypeStruct((NW, NCH, G), jnp.float32),
        ],
        scratch_types=[
            pltpu.VMEM((NIDX,), jnp.int32),
            pltpu.VMEM((NIDX,), jnp.int32),
            pltpu.VMEM((G, D), jnp.float32),
            pltpu.VMEM((G, D), jnp.float32),
            pltpu.VMEM((SPW, D), jnp.float32),
            pltpu.VMEM((NCH, G), jnp.float32),
            pltpu.SemaphoreType.DMA,
            pltpu.SemaphoreType.DMA,
            pltpu.SemaphoreType.DMA,
        ],
    )
    return f(idx, idx2, table, lin_tab)


def _mlp_body(inter_ref, lin_ref, w1_ref, b1_ref, w2_ref, b2_ref, w3_ref,
              c_ref, out_ref):
    inter = inter_ref[...]
    h = jnp.dot(inter, w1_ref[...], preferred_element_type=jnp.float32)
    h = jnp.maximum(h + b1_ref[...], 0.0)
    h = jnp.dot(h, w2_ref[...], preferred_element_type=jnp.float32)
    h = jnp.maximum(h + b2_ref[...], 0.0)
    deep = jnp.sum(h * w3_ref[...], axis=1, keepdims=True)
    lr = jnp.sum(lin_ref[...], axis=1, keepdims=True)
    out_ref[...] = jax.nn.sigmoid(deep + lr + c_ref[...])


@jax.jit
def _mlp(inter, lin2, W1, b1r, W2, b2r, w3r, c):
    blk = 2048
    return pl.pallas_call(
        _mlp_body,
        grid=(B // blk,),
        in_specs=[
            pl.BlockSpec((blk, D), lambda i: (i, 0)),
            pl.BlockSpec((blk, F), lambda i: (i, 0)),
            pl.BlockSpec((D, 256), lambda i: (0, 0)),
            pl.BlockSpec((1, 256), lambda i: (0, 0)),
            pl.BlockSpec((256, 128), lambda i: (0, 0)),
            pl.BlockSpec((1, 128), lambda i: (0, 0)),
            pl.BlockSpec((1, 128), lambda i: (0, 0)),
            pl.BlockSpec((1, 1), lambda i: (0, 0)),
        ],
        out_specs=pl.BlockSpec((blk, 1), lambda i: (i, 0)),
        out_shape=jax.ShapeDtypeStruct((B, 1), jnp.float32),
    )(inter, lin2, W1, b1r, W2, b2r, w3r, c)


def kernel(x, emb_linear, emb_table, bias, W1, b1, W2, b2, W3, b3):
    g = x.astype(jnp.int32)
    idx = g.reshape(NW, NIDX)
    v = g & (TBLK - 1)
    s = g - v + ((v & (TBLK // 2 - 1)) << 1) + (v >> (TBLK.bit_length() - 2))
    idx2 = s.reshape(NW, NIDX)
    t1, l1 = _repack(emb_table.T, emb_linear.T)
    tbl = t1.reshape(2 * ROWS_PAD, D)
    lin_tab = l1.reshape(VPAD)
    inter, lin_vals = _sc_gather(idx, idx2, tbl, lin_tab)
    lin2 = lin_vals.reshape(B, F)
    c = (b3 + bias).reshape(1, 1)
    return _mlp(inter, lin2, W1, b1.reshape(1, 256), W2,
                b2.reshape(1, 128), W3.reshape(1, 128), c)


# speedup vs baseline: 3.6973x; 1.0433x over previous
import jax
import jax.numpy as jnp
from jax import lax
from jax.experimental import pallas as pl
from jax.experimental.pallas import tpu as pltpu
from jax.experimental.pallas import tpu_sc as plsc

V = 1000000
D = 64
B = 16384
F = 26

NC = 2
NS = 16
NW = NC * NS
SPW = B // NW
CHUNK = 4
G = CHUNK * F
NCH = SPW // CHUNK
NIDX = NCH * G

TBLK = 32768
TGRID = (V + TBLK - 1) // TBLK
VPAD = TGRID * TBLK
ROWS_PAD = VPAD // 2


def _tr_body(tv_ref, lv_ref, out_ref, out2_ref):
    blk = tv_ref[...]
    tp = jnp.swapaxes(blk, 0, 1)
    out_ref[...] = jnp.concatenate(
        [tp[:TBLK // 2, :], tp[TBLK // 2:, :]], axis=1)
    lv = lv_ref[...]
    out2_ref[...] = jnp.concatenate(
        [lv[:, 128 * i:128 * (i + 1)] for i in range(TBLK // 128)], axis=0)


@jax.jit
def _repack(tv, lv):
    return pl.pallas_call(
        _tr_body,
        grid=(TGRID,),
        in_specs=[pl.BlockSpec((D, TBLK), lambda i: (0, i)),
                  pl.BlockSpec((1, TBLK), lambda i: (0, i))],
        out_specs=[pl.BlockSpec((TBLK // 2, 128), lambda i: (i, 0)),
                   pl.BlockSpec((TBLK // 128, 128), lambda i: (i, 0))],
        out_shape=[jax.ShapeDtypeStruct((ROWS_PAD, 128), jnp.float32),
                   jax.ShapeDtypeStruct((VPAD // 128, 128), jnp.float32)],
    )(tv, lv)


def _sc_body(idx_hbm, idx2_hbm, table_hbm, lin_hbm, out_inter, out_lin,
             idx_v, idx2_v, buf_a, buf_b, out_v, lin_acc,
             sem_a, sem_b, sem_l):
    wid = lax.axis_index("s") * NC + lax.axis_index("c")

    pltpu.sync_copy(idx_hbm.at[wid], idx_v)
    pltpu.sync_copy(idx2_hbm.at[wid], idx2_v)

    def row_copy(j, buf, sem):
        return pltpu.make_async_copy(
            table_hbm.at[idx2_v.at[pl.ds(j * G, G)]], buf, sem)

    def lin_copy(j):
        return pltpu.make_async_copy(
            lin_hbm.at[idx_v.at[pl.ds(j * G, G)]], lin_acc.at[j], sem_l)

    row_copy(0, buf_a, sem_a).start()
    lin_copy(0).start()
    row_copy(1, buf_b, sem_b).start()
    lin_copy(1).start()

    def step(t, carry):
        for slot, (buf, sem) in enumerate(((buf_a, sem_a), (buf_b, sem_b))):
            j = 2 * t + slot
            row_copy(j, buf, sem).wait()
            lin_copy(j).wait()
            for s in range(CHUNK):
                acc = [jnp.zeros((16,), jnp.float32) for _ in range(4)]
                accq = [jnp.zeros((16,), jnp.float32) for _ in range(4)]
                for r in range(F):
                    row = s * F + r
                    for c in range(4):
                        v = buf[row, pl.ds(16 * c, 16)]
                        acc[c] = acc[c] + v
                        accq[c] = accq[c] + v * v
                orow = CHUNK * j + s
                for c in range(4):
                    out_v[orow, pl.ds(16 * c, 16)] = 0.5 * (
                        acc[c] * acc[c] - accq[c])
            nxt = j + 2

            @pl.when(nxt < NCH)
            def _():
                row_copy(nxt, buf, sem).start()
                lin_copy(nxt).start()
        return carry

    lax.fori_loop(0, NCH // 2, step, 0)

    pltpu.sync_copy(out_v, out_inter.at[pl.ds(wid * SPW, SPW)])
    pltpu.sync_copy(lin_acc, out_lin.at[wid])


@jax.jit
def _sc_gather(idx, idx2, table, lin_tab):
    mesh = plsc.VectorSubcoreMesh(core_axis_name="c", subcore_axis_name="s")
    f = pl.kernel(
        _sc_body,
        mesh=mesh,
        compiler_params=pltpu.CompilerParams(use_tc_tiling_on_sc=False),
        out_type=[
            jax.ShapeDtypeStruct((B, D), jnp.float32),
            jax.ShapeDtypeStruct((NW, NCH, G), jnp.float32),
        ],
        scratch_types=[
            pltpu.VMEM((NIDX,), jnp.int32),
            pltpu.VMEM((NIDX,), jnp.int32),
            pltpu.VMEM((G, D), jnp.float32),
            pltpu.VMEM((G, D), jnp.float32),
            pltpu.VMEM((SPW, D), jnp.float32),
            pltpu.VMEM((NCH, G), jnp.float32),
            pltpu.SemaphoreType.DMA,
            pltpu.SemaphoreType.DMA,
            pltpu.SemaphoreType.DMA,
        ],
    )
    return f(idx, idx2, table, lin_tab)


def _mlp_body(inter_ref, lin_ref, w1_ref, b1_ref, w2_ref, b2_ref, w3_ref,
              c_ref, out_ref):
    inter = inter_ref[...]
    h = jnp.dot(inter, w1_ref[...], preferred_element_type=jnp.float32)
    h = jnp.maximum(h + b1_ref[...], 0.0)
    h = jnp.dot(h, w2_ref[...], preferred_element_type=jnp.float32)
    h = jnp.maximum(h + b2_ref[...], 0.0)
    deep = jnp.sum(h * w3_ref[...], axis=1, keepdims=True)
    lr = jnp.sum(lin_ref[...], axis=1, keepdims=True)
    out_ref[...] = jax.nn.sigmoid(deep + lr + c_ref[...])


@jax.jit
def _mlp(inter, lin2, W1, b1r, W2, b2r, w3r, c):
    blk = 2048
    return pl.pallas_call(
        _mlp_body,
        grid=(B // blk,),
        in_specs=[
            pl.BlockSpec((blk, D), lambda i: (i, 0)),
            pl.BlockSpec((blk, F), lambda i: (i, 0)),
            pl.BlockSpec((D, 256), lambda i: (0, 0)),
            pl.BlockSpec((1, 256), lambda i: (0, 0)),
            pl.BlockSpec((256, 128), lambda i: (0, 0)),
            pl.BlockSpec((1, 128), lambda i: (0, 0)),
            pl.BlockSpec((1, 128), lambda i: (0, 0)),
            pl.BlockSpec((1, 1), lambda i: (0, 0)),
        ],
        out_specs=pl.BlockSpec((blk, 1), lambda i: (i, 0)),
        out_shape=jax.ShapeDtypeStruct((B, 1), jnp.float32),
    )(inter, lin2, W1, b1r, W2, b2r, w3r, c)


def kernel(x, emb_linear, emb_table, bias, W1, b1, W2, b2, W3, b3):
    g = x.astype(jnp.int32)
    idx = g.reshape(NW, NIDX)
    v = g & (TBLK - 1)
    s = g - v + ((v & (TBLK // 2 - 1)) << 1) + (v >> (TBLK.bit_length() - 2))
    idx2 = s.reshape(NW, NIDX)
    t1, l1 = _repack(emb_table.T, emb_linear.T)
    tbl = t1.reshape(2 * ROWS_PAD, D)
    lin_tab = l1.reshape(VPAD)
    inter, lin_vals = _sc_gather(idx, idx2, tbl, lin_tab)
    lin2 = lin_vals.reshape(B, F)
    c = (b3 + bias).reshape(1, 1)
    return _mlp(inter, lin2, W1, b1.reshape(1, 256), W2,
                b2.reshape(1, 128), W3.reshape(1, 128), c)
